# baseline (speedup 1.0000x reference)
import functools

import jax
import jax.numpy as jnp
from jax import lax
from jax.experimental import pallas as pl
from jax.experimental.pallas import tpu as pltpu
from jax.experimental.pallas import tpu_sc as plsc

B, N, D, K = 2, 4096, 128, 16
R = 256
BN = B * N
BNK = BN * K
XP = 16
NEG = 3.0e38


def _knn_body(xyzf_ref, rows_ref, feat_ref, w1_ref, b1_ref, wd1_ref,
              out_ref, gidx_ref, p_ref):
    b = pl.program_id(0)
    xyz_f = xyzf_ref[0]
    rows = rows_ref[0]
    sqf = jnp.sum(xyz_f * xyz_f, axis=1, keepdims=True)
    lane_f = lax.broadcasted_iota(jnp.int32, (N, XP), 1)
    Y = jnp.where(lane_f == 3, sqf, xyz_f)
    lane_r = lax.broadcasted_iota(jnp.int32, (R, XP), 1)
    A = jnp.where(lane_r == 3, 1.0, -2.0 * rows)
    sq_col = jnp.sum(rows * rows, axis=1, keepdims=True)
    d2 = sq_col + lax.dot_general(A, Y, (((1,), (1,)), ((), ())),
                                  preferred_element_type=jnp.float32)

    iota_col = lax.broadcasted_iota(jnp.int32, (R, N), 1)
    bits = lax.bitcast_convert_type(jnp.maximum(d2, 0.0), jnp.int32)
    q = lax.bitcast_convert_type(
        jnp.bitwise_or(jnp.bitwise_and(bits, -4096), iota_col),
        jnp.float32)
    lane_k = lax.broadcasted_iota(jnp.int32, (R, K), 1)

    CH, CW, T = 32, 128, 6
    vals = q
    parts = []
    for t in range(T):
        mins_t = []
        newv = []
        for c in range(CH):
            sl = lax.slice_in_dim(vals, c * CW, (c + 1) * CW, axis=1)
            mc = jnp.min(sl, axis=1, keepdims=True)
            newv.append(jnp.where(sl == mc, NEG, sl))
            mins_t.append(mc)
        vals = jnp.concatenate(newv, axis=1)
        parts.append(jnp.concatenate(mins_t, axis=1))
    V = jnp.concatenate(parts, axis=1)

    idx_acc = jnp.zeros((R, K), jnp.int32)
    for k in range(K):
        m = jnp.min(V, axis=1, keepdims=True)
        am = jnp.bitwise_and(lax.bitcast_convert_type(m, jnp.int32), 4095)
        idx_acc = jnp.where(lane_k == k, am, idx_acc)
        V = jnp.where(V == m, NEG, V)

    consumed = V >= jnp.float32(1.0e38)
    allc = lax.slice_in_dim(consumed, 0, CH, axis=1)
    for t in range(1, T):
        allc = jnp.logical_and(
            allc, lax.slice_in_dim(consumed, t * CH, (t + 1) * CH, axis=1))
    need_fb = jnp.max(allc.astype(jnp.int32)) > 0

    @pl.when(jnp.logical_not(need_fb))
    def _fast():
        gidx_ref[0] = idx_acc + b * N

    @pl.when(need_fb)
    def _slow():
        qq = q
        acc = jnp.zeros((R, K), jnp.int32)
        for k in range(K):
            m = jnp.min(qq, axis=1, keepdims=True)
            am = jnp.bitwise_and(
                lax.bitcast_convert_type(m, jnp.int32), 4095)
            acc = jnp.where(lane_k == k, am, acc)
            qq = jnp.where(qq == m, NEG, qq)
        gidx_ref[0] = acc + b * N

    out_ref[0] = (jnp.dot(feat_ref[0], w1_ref[...],
                          preferred_element_type=jnp.float32)
                  + b1_ref[...])
    p_ref[0] = jnp.dot(rows, wd1_ref[...],
                       preferred_element_type=jnp.float32)


def _stage1(xyz16, feat, W1, b1, Wd1p):
    nb = xyz16.shape[0]
    nblk = N // R
    grid = (nb, nblk)
    return pl.pallas_call(
        _knn_body,
        grid=grid,
        in_specs=[
            pl.BlockSpec((1, N, XP), lambda b, i: (b, 0, 0)),
            pl.BlockSpec((1, R, XP), lambda b, i: (b, i, 0)),
            pl.BlockSpec((1, R, D), lambda b, i: (b, i, 0)),
            pl.BlockSpec((D, D), lambda b, i: (0, 0)),
            pl.BlockSpec((1, D), lambda b, i: (0, 0)),
            pl.BlockSpec((XP, D), lambda b, i: (0, 0)),
        ],
        out_specs=[
            pl.BlockSpec((1, R, D), lambda b, i: (b, i, 0)),
            pl.BlockSpec((1, R, K), lambda b, i: (b, i, 0)),
            pl.BlockSpec((1, R, D), lambda b, i: (b, i, 0)),
        ],
        out_shape=[
            jax.ShapeDtypeStruct((nb, N, D), jnp.float32),
            jax.ShapeDtypeStruct((nb, N, K), jnp.int32),
            jax.ShapeDtypeStruct((nb, N, D), jnp.float32),
        ],
    )(xyz16, xyz16, feat, W1, b1, Wd1p)


def _sc_gather(gidx3, feat_tab, p_tab):
    info = plsc.get_sparse_core_info()
    nc, ns = info.num_cores, info.num_subcores
    nw = nc * ns
    nrows = gidx3.shape[0] * gidx3.shape[1] * gidx3.shape[2]
    per_w = nrows // nw
    nchunk = per_w // 128
    mesh = plsc.VectorSubcoreMesh(core_axis_name="c", subcore_axis_name="s")

    @functools.partial(
        pl.kernel, mesh=mesh,
        out_type=[
            jax.ShapeDtypeStruct((nrows, D), jnp.float32),
            jax.ShapeDtypeStruct((nrows, D), jnp.float32),
        ],
        scratch_types=[
            pltpu.VMEM((nchunk, 128), jnp.int32),
            pltpu.VMEM((2, 128, D), jnp.float32),
            pltpu.VMEM((2, 128, D), jnp.float32),
            pltpu.SemaphoreType.DMA,
            pltpu.SemaphoreType.DMA,
            pltpu.SemaphoreType.DMA,
            pltpu.SemaphoreType.DMA,
        ],
    )
    def k(gidx_hbm, ftab_hbm, ptab_hbm, gfeat_hbm, gp_hbm,
          idx_v, fbuf, pbuf, sf0, sf1, sp0, sp1):
        wid = lax.axis_index("s") * nc + lax.axis_index("c")
        pltpu.sync_copy(gidx_hbm.at[wid], idx_v)
        base0 = wid * per_w
        sems = ((sf0, sp0), (sf1, sp1))

        def issue(c, s):
            sf, sp = sems[s]
            pltpu.async_copy(ftab_hbm.at[idx_v.at[c]], fbuf.at[s], sf)
            pltpu.async_copy(ptab_hbm.at[idx_v.at[c]], pbuf.at[s], sp)

        def drain_store(c, s):
            sf, sp = sems[s]
            pltpu.make_async_copy(ftab_hbm.at[idx_v.at[c]],
                                  fbuf.at[s], sf).wait()
            pltpu.make_async_copy(ptab_hbm.at[idx_v.at[c]],
                                  pbuf.at[s], sp).wait()
            base = base0 + c * 128
            pltpu.sync_copy(fbuf.at[s], gfeat_hbm.at[pl.ds(base, 128)])
            pltpu.sync_copy(pbuf.at[s], gp_hbm.at[pl.ds(base, 128)])

        issue(0, 0)

        def body(i, _):
            issue(2 * i + 1, 1)
            drain_store(2 * i, 0)

            @pl.when(i < nchunk // 2 - 1)
            def _issue_next():
                issue(2 * i + 2, 0)

            drain_store(2 * i + 1, 1)
            return _

        lax.fori_loop(0, nchunk // 2, body, None)

    return k(gidx3, feat_tab, p_tab)


def _attn_body(out_ref, feat_ref, p_ref, gfeat_ref, gp_ref,
               bd1_ref, wd2_ref, bd2_ref, wg_ref,
               wsg_ref, wpg_ref, bgam_ref, wa_ref, ba_ref,
               w2_ref, b2_ref, o_ref):
    g = gfeat_ref[...]
    gp = gp_ref[...]
    p = p_ref[...]
    e1 = ((p[:, None, :] - gp.reshape(R, K, D)).reshape(R * K, D)
          + bd1_ref[...])
    delta = jnp.maximum(
        jnp.dot(e1, wd2_ref[...], preferred_element_type=jnp.float32)
        + bd2_ref[...], 0.0)
    dwg = jnp.dot(delta, wg_ref[...], preferred_element_type=jnp.float32)
    psi_t = jnp.dot(g, wsg_ref[...], preferred_element_type=jnp.float32)
    phi_t = jnp.dot(out_ref[...], wpg_ref[...],
                    preferred_element_type=jnp.float32)
    gamma = (phi_t[:, None, :] - psi_t.reshape(R, K, D)
             + dwg.reshape(R, K, D) + bgam_ref[...])
    mx = jnp.max(gamma, axis=2, keepdims=True)
    ex = jnp.exp(gamma - mx)
    rho = ex / jnp.sum(ex, axis=2, keepdims=True)
    alpha = (jnp.dot(g, wa_ref[...], preferred_element_type=jnp.float32)
             + ba_ref[...] + delta).reshape(R, K, D)
    agg = jnp.sum(rho * alpha, axis=1)
    o_ref[...] = (jnp.dot(agg, w2_ref[...],
                          preferred_element_type=jnp.float32)
                  + b2_ref[...] + feat_ref[...])


def _stage3(out2, feat2, p2, g_feat, g_p, weights):
    nblk = out2.shape[0] // R
    wspecs = []
    for w in weights:
        wspecs.append(pl.BlockSpec(w.shape, lambda i, nd=w.ndim: (0,) * nd))
    return pl.pallas_call(
        _attn_body,
        grid=(nblk,),
        in_specs=[
            pl.BlockSpec((R, D), lambda i: (i, 0)),
            pl.BlockSpec((R, D), lambda i: (i, 0)),
            pl.BlockSpec((R, D), lambda i: (i, 0)),
            pl.BlockSpec((R * K, D), lambda i: (i, 0)),
            pl.BlockSpec((R * K, D), lambda i: (i, 0)),
        ] + wspecs,
        out_specs=pl.BlockSpec((R, D), lambda i: (i, 0)),
        out_shape=jax.ShapeDtypeStruct((out2.shape[0], D), jnp.float32),
    )(out2, feat2, p2, g_feat, g_p, *weights)


def kernel(points_xyz, points_features, W1, b1, Wg, bg, Wphi, bphi,
           Wpsi, bpsi, Wa, ba, Wd1, bd1, Wd2, bd2, W2, b2):
    f32 = jnp.float32
    xyz16 = jnp.pad(points_xyz, ((0, 0), (0, 0), (0, XP - 3)))
    Wd1p = jnp.pad(Wd1, ((0, XP - 3), (0, 0)))

    Wsg = jnp.dot(Wpsi, Wg, preferred_element_type=f32)
    Wpg = jnp.dot(Wphi, Wg, preferred_element_type=f32)
    bgam = (jnp.dot((bphi - bpsi).reshape(1, D), Wg,
                    preferred_element_type=f32) + bg.reshape(1, D))
    weights = [
        bd1.reshape(1, D), Wd2, bd2.reshape(1, D), Wg,
        Wsg, Wpg, bgam, Wa, ba.reshape(1, D),
        W2, b2.reshape(1, D),
    ]

    outs = []
    for b in range(B):
        xyz_b = lax.slice_in_dim(xyz16, b, b + 1, axis=0)
        feat_b = lax.slice_in_dim(points_features, b, b + 1, axis=0)
        out, gidx, p = _stage1(xyz_b, feat_b, W1, b1.reshape(1, D), Wd1p)
        out2 = out.reshape(N, D)
        p2 = p.reshape(N, D)
        gidx3 = gidx.reshape(32, N * K // 32 // 128, 128)
        g_feat, g_p = _sc_gather(gidx3, out2, p2)
        feat2 = feat_b.reshape(N, D)
        o = _stage3(out2, feat2, p2, g_feat, g_p, weights)
        outs.append(o.reshape(1, N, D))
    return (points_xyz, jnp.concatenate(outs, axis=0))


# speedup vs baseline: 1.0814x; 1.0814x over previous
import functools

import jax
import jax.numpy as jnp
from jax import lax
from jax.experimental import pallas as pl
from jax.experimental.pallas import tpu as pltpu
from jax.experimental.pallas import tpu_sc as plsc

B, N, D, K = 2, 4096, 128, 16
R = 256
BN = B * N
BNK = BN * K
XP = 16
NEG = 3.0e38


def _knn_body(xyzf_ref, rows_ref, feat_ref, w1_ref, b1_ref, wd1_ref,
              out_ref, gidx_ref, p_ref):
    b = pl.program_id(0)
    xyz_f = xyzf_ref[0]
    rows = rows_ref[0]
    sqf = jnp.sum(xyz_f * xyz_f, axis=1, keepdims=True)
    lane_f = lax.broadcasted_iota(jnp.int32, (N, XP), 1)
    Y = jnp.where(lane_f == 3, sqf, xyz_f)
    lane_r = lax.broadcasted_iota(jnp.int32, (R, XP), 1)
    A = jnp.where(lane_r == 3, 1.0, -2.0 * rows)
    sq_col = jnp.sum(rows * rows, axis=1, keepdims=True)
    d2 = sq_col + lax.dot_general(A, Y, (((1,), (1,)), ((), ())),
                                  preferred_element_type=jnp.float32)

    iota_col = lax.broadcasted_iota(jnp.int32, (R, N), 1)
    bits = lax.bitcast_convert_type(jnp.maximum(d2, 0.0), jnp.int32)
    q = lax.bitcast_convert_type(
        jnp.bitwise_or(jnp.bitwise_and(bits, -4096), iota_col),
        jnp.float32)
    lane_k = lax.broadcasted_iota(jnp.int32, (R, K), 1)
    idx_acc = jnp.zeros((R, K), jnp.int32)
    for k in range(K):
        m = jnp.min(q, axis=1, keepdims=True)
        am = jnp.bitwise_and(lax.bitcast_convert_type(m, jnp.int32), 4095)
        idx_acc = jnp.where(lane_k == k, am, idx_acc)
        q = jnp.where(q == m, NEG, q)
    gidx_ref[0] = idx_acc + b * N

    out_ref[0] = (jnp.dot(feat_ref[0], w1_ref[...],
                          preferred_element_type=jnp.float32)
                  + b1_ref[...])
    p_ref[0] = jnp.dot(rows, wd1_ref[...],
                       preferred_element_type=jnp.float32)


def _stage1(xyz16, feat, W1, b1, Wd1p):
    nb = xyz16.shape[0]
    nblk = N // R
    grid = (nb, nblk)
    return pl.pallas_call(
        _knn_body,
        grid=grid,
        in_specs=[
            pl.BlockSpec((1, N, XP), lambda b, i: (b, 0, 0)),
            pl.BlockSpec((1, R, XP), lambda b, i: (b, i, 0)),
            pl.BlockSpec((1, R, D), lambda b, i: (b, i, 0)),
            pl.BlockSpec((D, D), lambda b, i: (0, 0)),
            pl.BlockSpec((1, D), lambda b, i: (0, 0)),
            pl.BlockSpec((XP, D), lambda b, i: (0, 0)),
        ],
        out_specs=[
            pl.BlockSpec((1, R, D), lambda b, i: (b, i, 0)),
            pl.BlockSpec((1, R, K), lambda b, i: (b, i, 0)),
            pl.BlockSpec((1, R, D), lambda b, i: (b, i, 0)),
        ],
        out_shape=[
            jax.ShapeDtypeStruct((nb, N, D), jnp.float32),
            jax.ShapeDtypeStruct((nb, N, K), jnp.int32),
            jax.ShapeDtypeStruct((nb, N, D), jnp.float32),
        ],
    )(xyz16, xyz16, feat, W1, b1, Wd1p)


def _sc_gather(gidx3, feat_tab, p_tab):
    info = plsc.get_sparse_core_info()
    nc, ns = info.num_cores, info.num_subcores
    nw = nc * ns
    nrows = gidx3.shape[0] * gidx3.shape[1] * gidx3.shape[2]
    per_w = nrows // nw
    nchunk = per_w // 128
    mesh = plsc.VectorSubcoreMesh(core_axis_name="c", subcore_axis_name="s")

    @functools.partial(
        pl.kernel, mesh=mesh,
        out_type=[
            jax.ShapeDtypeStruct((nrows, D), jnp.float32),
            jax.ShapeDtypeStruct((nrows, D), jnp.float32),
        ],
        scratch_types=[
            pltpu.VMEM((nchunk, 128), jnp.int32),
            pltpu.VMEM((2, 128, D), jnp.float32),
            pltpu.VMEM((2, 128, D), jnp.float32),
            pltpu.SemaphoreType.DMA,
            pltpu.SemaphoreType.DMA,
            pltpu.SemaphoreType.DMA,
            pltpu.SemaphoreType.DMA,
        ],
    )
    def k(gidx_hbm, ftab_hbm, ptab_hbm, gfeat_hbm, gp_hbm,
          idx_v, fbuf, pbuf, sf0, sf1, sp0, sp1):
        wid = lax.axis_index("s") * nc + lax.axis_index("c")
        pltpu.sync_copy(gidx_hbm.at[wid], idx_v)
        base0 = wid * per_w
        sems = ((sf0, sp0), (sf1, sp1))

        def issue(c, s):
            sf, sp = sems[s]
            pltpu.async_copy(ftab_hbm.at[idx_v.at[c]], fbuf.at[s], sf)
            pltpu.async_copy(ptab_hbm.at[idx_v.at[c]], pbuf.at[s], sp)

        def drain_store(c, s):
            sf, sp = sems[s]
            pltpu.make_async_copy(ftab_hbm.at[idx_v.at[c]],
                                  fbuf.at[s], sf).wait()
            pltpu.make_async_copy(ptab_hbm.at[idx_v.at[c]],
                                  pbuf.at[s], sp).wait()
            base = base0 + c * 128
            pltpu.sync_copy(fbuf.at[s], gfeat_hbm.at[pl.ds(base, 128)])
            pltpu.sync_copy(pbuf.at[s], gp_hbm.at[pl.ds(base, 128)])

        issue(0, 0)

        def body(i, _):
            issue(2 * i + 1, 1)
            drain_store(2 * i, 0)

            @pl.when(i < nchunk // 2 - 1)
            def _issue_next():
                issue(2 * i + 2, 0)

            drain_store(2 * i + 1, 1)
            return _

        lax.fori_loop(0, nchunk // 2, body, None)

    return k(gidx3, feat_tab, p_tab)


def _attn_body(out_ref, feat_ref, p_ref, gfeat_ref, gp_ref,
               bd1_ref, wd2_ref, bd2_ref, wg_ref,
               wsg_ref, wpg_ref, bgam_ref, wa_ref, ba_ref,
               w2_ref, b2_ref, o_ref):
    g = gfeat_ref[...]
    gp = gp_ref[...]
    p = p_ref[...]
    e1 = ((p[:, None, :] - gp.reshape(R, K, D)).reshape(R * K, D)
          + bd1_ref[...])
    delta = jnp.maximum(
        jnp.dot(e1, wd2_ref[...], preferred_element_type=jnp.float32)
        + bd2_ref[...], 0.0)
    dwg = jnp.dot(delta, wg_ref[...], preferred_element_type=jnp.float32)
    psi_t = jnp.dot(g, wsg_ref[...], preferred_element_type=jnp.float32)
    phi_t = jnp.dot(out_ref[...], wpg_ref[...],
                    preferred_element_type=jnp.float32)
    gamma = (phi_t[:, None, :] - psi_t.reshape(R, K, D)
             + dwg.reshape(R, K, D) + bgam_ref[...])
    mx = jnp.max(gamma, axis=2, keepdims=True)
    ex = jnp.exp(gamma - mx)
    rho = ex / jnp.sum(ex, axis=2, keepdims=True)
    alpha = (jnp.dot(g, wa_ref[...], preferred_element_type=jnp.float32)
             + ba_ref[...] + delta).reshape(R, K, D)
    agg = jnp.sum(rho * alpha, axis=1)
    o_ref[...] = (jnp.dot(agg, w2_ref[...],
                          preferred_element_type=jnp.float32)
                  + b2_ref[...] + feat_ref[...])


def _stage3(out2, feat2, p2, g_feat, g_p, weights):
    nblk = out2.shape[0] // R
    wspecs = []
    for w in weights:
        wspecs.append(pl.BlockSpec(w.shape, lambda i, nd=w.ndim: (0,) * nd))
    return pl.pallas_call(
        _attn_body,
        grid=(nblk,),
        in_specs=[
            pl.BlockSpec((R, D), lambda i: (i, 0)),
            pl.BlockSpec((R, D), lambda i: (i, 0)),
            pl.BlockSpec((R, D), lambda i: (i, 0)),
            pl.BlockSpec((R * K, D), lambda i: (i, 0)),
            pl.BlockSpec((R * K, D), lambda i: (i, 0)),
        ] + wspecs,
        out_specs=pl.BlockSpec((R, D), lambda i: (i, 0)),
        out_shape=jax.ShapeDtypeStruct((out2.shape[0], D), jnp.float32),
    )(out2, feat2, p2, g_feat, g_p, *weights)


def kernel(points_xyz, points_features, W1, b1, Wg, bg, Wphi, bphi,
           Wpsi, bpsi, Wa, ba, Wd1, bd1, Wd2, bd2, W2, b2):
    f32 = jnp.float32
    xyz16 = jnp.pad(points_xyz, ((0, 0), (0, 0), (0, XP - 3)))
    Wd1p = jnp.pad(Wd1, ((0, XP - 3), (0, 0)))

    Wsg = jnp.dot(Wpsi, Wg, preferred_element_type=f32)
    Wpg = jnp.dot(Wphi, Wg, preferred_element_type=f32)
    bgam = (jnp.dot((bphi - bpsi).reshape(1, D), Wg,
                    preferred_element_type=f32) + bg.reshape(1, D))
    weights = [
        bd1.reshape(1, D), Wd2, bd2.reshape(1, D), Wg,
        Wsg, Wpg, bgam, Wa, ba.reshape(1, D),
        W2, b2.reshape(1, D),
    ]

    outs = []
    for b in range(B):
        xyz_b = lax.slice_in_dim(xyz16, b, b + 1, axis=0)
        feat_b = lax.slice_in_dim(points_features, b, b + 1, axis=0)
        out, gidx, p = _stage1(xyz_b, feat_b, W1, b1.reshape(1, D), Wd1p)
        out2 = out.reshape(N, D)
        p2 = p.reshape(N, D)
        gidx3 = gidx.reshape(32, N * K // 32 // 128, 128)
        g_feat, g_p = _sc_gather(gidx3, out2, p2)
        feat2 = feat_b.reshape(N, D)
        o = _stage3(out2, feat2, p2, g_feat, g_p, weights)
        outs.append(o.reshape(1, N, D))
    return (points_xyz, jnp.concatenate(outs, axis=0))
